# baseline (speedup 1.0000x reference)
import functools

import numpy as np

import jax
import jax.numpy as jnp
from jax import lax
from jax.experimental import pallas as pl
from jax.experimental.pallas import tpu as pltpu
from jax.experimental.pallas import tpu_sc as plsc

N = 10000
E = 320000
D = 128

NC = 2
NS = 16
NW = NC * NS
K = 128
CHUNKS = 80
EPW = CHUNKS * K
EPAD = NW * EPW
GRP = 8
NPAD = 10240
RPT = NPAD // NS

_mesh = plsc.VectorSubcoreMesh(
    core_axis_name="c", subcore_axis_name="s", num_cores=NC, num_subcores=NS
)


@functools.partial(
    pl.kernel,
    out_type=jax.ShapeDtypeStruct((NC * NPAD,), jnp.float32),
    mesh=_mesh,
    scratch_types=[
        pltpu.VMEM((CHUNKS, K), jnp.int32),
        pltpu.VMEM((K,), jnp.float32),
        pltpu.VMEM((RPT,), jnp.float32),
        pltpu.VMEM_SHARED((NPAD,), jnp.float32),
    ],
)
def _deg_kernel(e3_hbm, out_hbm, dst_v, ones_v, zeros_v, acc_sh):
    c = lax.axis_index("c")
    s = lax.axis_index("s")
    wid = s * NC + c
    for i in range(K // 16):
        ones_v[pl.ds(16 * i, 16)] = jnp.ones((16,), jnp.float32)
    for i in range(RPT // 16):
        zeros_v[pl.ds(16 * i, 16)] = jnp.zeros((16,), jnp.float32)
    pltpu.sync_copy(zeros_v, acc_sh.at[pl.ds(s * RPT, RPT)])
    pltpu.sync_copy(e3_hbm.at[1].at[wid], dst_v)
    plsc.subcore_barrier()

    def body(j, carry):
        pltpu.sync_copy(ones_v, acc_sh.at[dst_v.at[j]], add=True)
        return carry

    lax.fori_loop(0, CHUNKS, body, 0)
    plsc.subcore_barrier()
    pltpu.sync_copy(acc_sh.at[pl.ds(s * RPT, RPT)],
                    out_hbm.at[pl.ds(c * NPAD + s * RPT, RPT)])


NGRP = CHUNKS // GRP


@functools.partial(
    pl.kernel,
    out_type=jax.ShapeDtypeStruct((NC, NPAD, D), jnp.float32),
    mesh=_mesh,
    scratch_types=[
        pltpu.VMEM((CHUNKS, K), jnp.int32),
        pltpu.VMEM((2, GRP, K), jnp.int32),
        pltpu.VMEM((K, D), jnp.float32),
        pltpu.VMEM((K, D), jnp.float32),
        pltpu.VMEM_SHARED((NPAD, D), jnp.float32),
        pltpu.SemaphoreType.DMA,
        pltpu.SemaphoreType.DMA,
        pltpu.SemaphoreType.DMA,
        pltpu.SemaphoreType.DMA,
    ],
)
def _agg_kernel(g_hbm, e3_hbm, out_hbm, src_v, dstb, buf0, buf1,
                acc_sh, gsem0, gsem1, ssem0, ssem1):
    bufs = (buf0, buf1)
    gsem = (gsem0, gsem1)
    ssem = (ssem0, ssem1)
    c = lax.axis_index("c")
    s = lax.axis_index("s")
    wid = s * NC + c

    def zrow(r, carry):
        for q in range(D // 16):
            buf0[r, pl.ds(16 * q, 16)] = jnp.zeros((16,), jnp.float32)
        return carry

    lax.fori_loop(0, K, zrow, 0)
    for p in range(RPT // K):
        pltpu.sync_copy(buf0, acc_sh.at[pl.ds(s * RPT + p * K, K)])
    pltpu.sync_copy(e3_hbm.at[0].at[wid], src_v)
    pltpu.sync_copy(e3_hbm.at[1].at[wid].at[pl.ds(0, GRP)], dstb.at[0])
    plsc.subcore_barrier()

    pltpu.async_copy(g_hbm.at[src_v.at[0]], bufs[0], gsem[0])
    for j in range(CHUNKS):
        slot = j % 2
        nslot = 1 - slot
        g = j // GRP
        jj = j % GRP
        if j + 1 < CHUNKS:
            if j >= 1:
                pj = j - 1
                pltpu.make_async_copy(
                    bufs[nslot], acc_sh.at[dstb.at[(pj // GRP) % 2].at[pj % GRP]],
                    ssem[nslot]).wait()
            pltpu.async_copy(g_hbm.at[src_v.at[j + 1]], bufs[nslot],
                             gsem[nslot])
        if jj == 1 and g + 1 < NGRP:
            pltpu.sync_copy(e3_hbm.at[1].at[wid].at[pl.ds((g + 1) * GRP, GRP)],
                            dstb.at[(g + 1) % 2])
        pltpu.make_async_copy(g_hbm.at[src_v.at[j]], bufs[slot],
                              gsem[slot]).wait()
        pltpu.async_copy(bufs[slot], acc_sh.at[dstb.at[g % 2].at[jj]],
                         ssem[slot], add=True)
    for b in range(2):
        j = CHUNKS - 2 + b
        pltpu.make_async_copy(
            bufs[j % 2], acc_sh.at[dstb.at[(j // GRP) % 2].at[j % GRP]],
            ssem[j % 2]).wait()
    plsc.subcore_barrier()
    pltpu.sync_copy(acc_sh.at[pl.ds(s * RPT, RPT)], out_hbm.at[c].at[pl.ds(s * RPT, RPT)])


BN = 2000


def _mm_body(x_ref, w_ref, h_ref):
    h_ref[...] = jnp.dot(x_ref[...], w_ref[...],
                         preferred_element_type=jnp.float32)


_mm = pl.pallas_call(
    _mm_body,
    grid=(N // BN,),
    in_specs=[
        pl.BlockSpec((BN, D), lambda i: (i, 0)),
        pl.BlockSpec((D, D), lambda i: (0, 0)),
    ],
    out_specs=pl.BlockSpec((BN, D), lambda i: (i, 0)),
    out_shape=jax.ShapeDtypeStruct((N, D), jnp.float32),
    compiler_params=pltpu.CompilerParams(dimension_semantics=("parallel",)),
)


def _scale_body(deg2_ref, h_ref, g_ref, dinv_ref):
    deg = deg2_ref[...].sum(axis=0) + 1.0
    dinv = lax.rsqrt(deg)
    g_ref[...] = h_ref[...] * dinv
    dinv_ref[...] = dinv


_scale = pl.pallas_call(
    _scale_body,
    grid=(N // BN,),
    in_specs=[
        pl.BlockSpec((NC, BN, 1), lambda i: (0, i, 0)),
        pl.BlockSpec((BN, D), lambda i: (i, 0)),
    ],
    out_specs=[
        pl.BlockSpec((BN, D), lambda i: (i, 0)),
        pl.BlockSpec((BN, 1), lambda i: (i, 0)),
    ],
    out_shape=[
        jax.ShapeDtypeStruct((N, D), jnp.float32),
        jax.ShapeDtypeStruct((N, 1), jnp.float32),
    ],
    compiler_params=pltpu.CompilerParams(dimension_semantics=("parallel",)),
)


def _fin_body(acc_ref, g_ref, dinv_ref, b_ref, o_ref):
    ssum = acc_ref[0] + acc_ref[1] + g_ref[...]
    o_ref[...] = jnp.maximum(ssum * dinv_ref[...] + b_ref[...], 0.0)


_fin = pl.pallas_call(
    _fin_body,
    grid=(N // BN,),
    in_specs=[
        pl.BlockSpec((NC, BN, D), lambda i: (0, i, 0)),
        pl.BlockSpec((BN, D), lambda i: (i, 0)),
        pl.BlockSpec((BN, 1), lambda i: (i, 0)),
        pl.BlockSpec((1, D), lambda i: (0, 0)),
    ],
    out_specs=pl.BlockSpec((BN, D), lambda i: (i, 0)),
    out_shape=jax.ShapeDtypeStruct((N, D), jnp.float32),
    compiler_params=pltpu.CompilerParams(dimension_semantics=("parallel",)),
)

_PAD_EDGES = np.stack([
    np.arange(EPAD - E, dtype=np.int32) % N,
    N + np.arange(EPAD - E, dtype=np.int32) % (NPAD - N),
])


def kernel(x, edge_index, cache_name, W, b):
    e3 = jnp.concatenate(
        [edge_index.astype(jnp.int32), jnp.asarray(_PAD_EDGES)], axis=1
    ).reshape(2, NW, CHUNKS, K)
    h = _mm(x, W)
    degp = _deg_kernel(e3).reshape(NC, NPAD, 1)
    g, dinv = _scale(degp, h)
    accp = _agg_kernel(g, e3)
    return _fin(accp, g, dinv, b.reshape(1, D))


# speedup vs baseline: 1.0869x; 1.0869x over previous
import functools

import numpy as np

import jax
import jax.numpy as jnp
from jax import lax
from jax.experimental import pallas as pl
from jax.experimental.pallas import tpu as pltpu
from jax.experimental.pallas import tpu_sc as plsc

N = 10000
E = 320000
D = 128

NC = 2
NS = 16
NW = NC * NS
K = 128
CHUNKS = 80
EPW = CHUNKS * K
EPAD = NW * EPW
GRP = 8
NPAD = 10240
RPT = NPAD // NS

_mesh = plsc.VectorSubcoreMesh(
    core_axis_name="c", subcore_axis_name="s", num_cores=NC, num_subcores=NS
)


@functools.partial(
    pl.kernel,
    out_type=jax.ShapeDtypeStruct((NC * NPAD,), jnp.float32),
    mesh=_mesh,
    scratch_types=[
        pltpu.VMEM((CHUNKS, K), jnp.int32),
        pltpu.VMEM((K,), jnp.float32),
        pltpu.VMEM((RPT,), jnp.float32),
        pltpu.VMEM_SHARED((NPAD,), jnp.float32),
        pltpu.SemaphoreType.DMA,
    ],
)
def _deg_kernel(e3_hbm, out_hbm, dst_v, ones_v, zeros_v, acc_sh, sem):
    c = lax.axis_index("c")
    s = lax.axis_index("s")
    wid = s * NC + c
    for i in range(K // 16):
        ones_v[pl.ds(16 * i, 16)] = jnp.ones((16,), jnp.float32)
    for i in range(RPT // 16):
        zeros_v[pl.ds(16 * i, 16)] = jnp.zeros((16,), jnp.float32)
    pltpu.sync_copy(zeros_v, acc_sh.at[pl.ds(s * RPT, RPT)])
    pltpu.sync_copy(e3_hbm.at[1].at[wid], dst_v)
    plsc.subcore_barrier()

    def fire(j, carry):
        pltpu.async_copy(ones_v, acc_sh.at[dst_v.at[j]], sem, add=True)
        return carry

    def drain(j, carry):
        pltpu.make_async_copy(ones_v, acc_sh.at[dst_v.at[j]], sem).wait()
        return carry

    lax.fori_loop(0, CHUNKS, fire, 0)
    lax.fori_loop(0, CHUNKS, drain, 0)
    plsc.subcore_barrier()
    pltpu.sync_copy(acc_sh.at[pl.ds(s * RPT, RPT)],
                    out_hbm.at[pl.ds(c * NPAD + s * RPT, RPT)])


NGRP = CHUNKS // GRP


@functools.partial(
    pl.kernel,
    out_type=jax.ShapeDtypeStruct((NC, NPAD, D), jnp.float32),
    mesh=_mesh,
    scratch_types=[
        pltpu.VMEM((CHUNKS, K), jnp.int32),
        pltpu.VMEM((2, GRP, K), jnp.int32),
        pltpu.VMEM((K, D), jnp.float32),
        pltpu.VMEM((K, D), jnp.float32),
        pltpu.VMEM_SHARED((NPAD, D), jnp.float32),
        pltpu.SemaphoreType.DMA,
        pltpu.SemaphoreType.DMA,
        pltpu.SemaphoreType.DMA,
        pltpu.SemaphoreType.DMA,
    ],
)
def _agg_kernel(g_hbm, e3_hbm, out_hbm, src_v, dstb, buf0, buf1,
                acc_sh, gsem0, gsem1, ssem0, ssem1):
    bufs = (buf0, buf1)
    gsem = (gsem0, gsem1)
    ssem = (ssem0, ssem1)
    c = lax.axis_index("c")
    s = lax.axis_index("s")
    wid = s * NC + c

    def zrow(r, carry):
        for q in range(D // 16):
            buf0[r, pl.ds(16 * q, 16)] = jnp.zeros((16,), jnp.float32)
        return carry

    lax.fori_loop(0, K, zrow, 0)
    for p in range(RPT // K):
        pltpu.sync_copy(buf0, acc_sh.at[pl.ds(s * RPT + p * K, K)])
    pltpu.sync_copy(e3_hbm.at[0].at[wid], src_v)
    pltpu.sync_copy(e3_hbm.at[1].at[wid].at[pl.ds(0, GRP)], dstb.at[0])
    plsc.subcore_barrier()

    pltpu.async_copy(g_hbm.at[src_v.at[0]], bufs[0], gsem[0])
    for j in range(CHUNKS):
        slot = j % 2
        nslot = 1 - slot
        g = j // GRP
        jj = j % GRP
        if j + 1 < CHUNKS:
            if j >= 1:
                pj = j - 1
                pltpu.make_async_copy(
                    bufs[nslot], acc_sh.at[dstb.at[(pj // GRP) % 2].at[pj % GRP]],
                    ssem[nslot]).wait()
            pltpu.async_copy(g_hbm.at[src_v.at[j + 1]], bufs[nslot],
                             gsem[nslot])
        if jj == 1 and g + 1 < NGRP:
            pltpu.sync_copy(e3_hbm.at[1].at[wid].at[pl.ds((g + 1) * GRP, GRP)],
                            dstb.at[(g + 1) % 2])
        pltpu.make_async_copy(g_hbm.at[src_v.at[j]], bufs[slot],
                              gsem[slot]).wait()
        pltpu.async_copy(bufs[slot], acc_sh.at[dstb.at[g % 2].at[jj]],
                         ssem[slot], add=True)
    for b in range(2):
        j = CHUNKS - 2 + b
        pltpu.make_async_copy(
            bufs[j % 2], acc_sh.at[dstb.at[(j // GRP) % 2].at[j % GRP]],
            ssem[j % 2]).wait()
    plsc.subcore_barrier()
    pltpu.sync_copy(acc_sh.at[pl.ds(s * RPT, RPT)], out_hbm.at[c].at[pl.ds(s * RPT, RPT)])


BN = 2000


def _mm_body(x_ref, w_ref, h_ref):
    h_ref[...] = jnp.dot(x_ref[...], w_ref[...],
                         preferred_element_type=jnp.float32)


_mm = pl.pallas_call(
    _mm_body,
    grid=(N // BN,),
    in_specs=[
        pl.BlockSpec((BN, D), lambda i: (i, 0)),
        pl.BlockSpec((D, D), lambda i: (0, 0)),
    ],
    out_specs=pl.BlockSpec((BN, D), lambda i: (i, 0)),
    out_shape=jax.ShapeDtypeStruct((N, D), jnp.float32),
    compiler_params=pltpu.CompilerParams(dimension_semantics=("parallel",)),
)


def _scale_body(deg2_ref, h_ref, g_ref, dinv_ref):
    deg = deg2_ref[...].sum(axis=1, keepdims=True) + 1.0
    dinv = lax.rsqrt(deg)
    g_ref[...] = h_ref[...] * dinv
    dinv_ref[...] = dinv


_scale = pl.pallas_call(
    _scale_body,
    grid=(N // BN,),
    in_specs=[
        pl.BlockSpec((BN, NC), lambda i: (i, 0)),
        pl.BlockSpec((BN, D), lambda i: (i, 0)),
    ],
    out_specs=[
        pl.BlockSpec((BN, D), lambda i: (i, 0)),
        pl.BlockSpec((BN, 1), lambda i: (i, 0)),
    ],
    out_shape=[
        jax.ShapeDtypeStruct((N, D), jnp.float32),
        jax.ShapeDtypeStruct((N, 1), jnp.float32),
    ],
    compiler_params=pltpu.CompilerParams(dimension_semantics=("parallel",)),
)


def _fin_body(acc_ref, g_ref, dinv_ref, b_ref, o_ref):
    ssum = acc_ref[0] + acc_ref[1] + g_ref[...]
    o_ref[...] = jnp.maximum(ssum * dinv_ref[...] + b_ref[...], 0.0)


_fin = pl.pallas_call(
    _fin_body,
    grid=(N // BN,),
    in_specs=[
        pl.BlockSpec((NC, BN, D), lambda i: (0, i, 0)),
        pl.BlockSpec((BN, D), lambda i: (i, 0)),
        pl.BlockSpec((BN, 1), lambda i: (i, 0)),
        pl.BlockSpec((1, D), lambda i: (0, 0)),
    ],
    out_specs=pl.BlockSpec((BN, D), lambda i: (i, 0)),
    out_shape=jax.ShapeDtypeStruct((N, D), jnp.float32),
    compiler_params=pltpu.CompilerParams(dimension_semantics=("parallel",)),
)

_PAD_EDGES = np.stack([
    np.arange(EPAD - E, dtype=np.int32) % N,
    N + np.arange(EPAD - E, dtype=np.int32) % (NPAD - N),
])


def kernel(x, edge_index, cache_name, W, b):
    e3 = jnp.concatenate(
        [edge_index.astype(jnp.int32), jnp.asarray(_PAD_EDGES)], axis=1
    ).reshape(2, NW, CHUNKS, K)
    h = _mm(x, W)
    degp = _deg_kernel(e3).reshape(NC, NPAD)
    g, dinv = _scale(degp[:, :N].T, h)
    accp = _agg_kernel(g, e3)
    return _fin(accp, g, dinv, b.reshape(1, D))
